# baseline (speedup 1.0000x reference)
import jax
import jax.numpy as jnp
from jax.experimental import pallas as pl
from jax.experimental.pallas import tpu as pltpu

E = 8
C = 512
D_IN = 2048
D_OUT = 1024
B = 1
S = 2048
T = B * S
MCH = 4
MB = T // MCH


def _fused_kernel(x_ref, cw_ref, w_ref, b_ref, out_ref, t_s):
    i = pl.program_id(0)

    def produce():
        x = x_ref[0].astype(jnp.bfloat16)
        w = w_ref[0].astype(jnp.bfloat16)
        t = jnp.dot(x, w, preferred_element_type=jnp.float32)
        t_s[i % 2] = (t + b_ref[0]).astype(jnp.bfloat16)

    def combine():
        tprev = t_s[(i + 1) % 2]
        for m in range(MCH):
            cw_m = cw_ref[m * MB:(m + 1) * MB, :].astype(jnp.bfloat16)
            out_ref[m * MB:(m + 1) * MB, :] += jnp.dot(
                cw_m, tprev, preferred_element_type=jnp.float32)

    @pl.when(i == 0)
    def _first():
        out_ref[...] = jnp.zeros_like(out_ref)
        produce()

    @pl.when(jnp.logical_and(i > 0, i < E))
    def _mid():
        produce()
        combine()

    @pl.when(i == E)
    def _last():
        combine()


def kernel(inputs, combine_weights, weight, bias):
    b = bias.reshape(E, 1, D_OUT)

    out = pl.pallas_call(
        _fused_kernel,
        grid=(E + 1,),
        in_specs=[
            pl.BlockSpec((1, C, D_IN), lambda i: (jnp.minimum(i, E - 1), 0, 0)),
            pl.BlockSpec((T, C), lambda i: (0, jnp.maximum(i - 1, 0))),
            pl.BlockSpec((1, D_IN, D_OUT), lambda i: (jnp.minimum(i, E - 1), 0, 0)),
            pl.BlockSpec((1, 1, D_OUT), lambda i: (jnp.minimum(i, E - 1), 0, 0)),
        ],
        out_specs=pl.BlockSpec((T, D_OUT), lambda i: (0, 0)),
        out_shape=jax.ShapeDtypeStruct((T, D_OUT), jnp.float32),
        scratch_shapes=[pltpu.VMEM((2, C, D_OUT), jnp.bfloat16)],
    )(inputs, combine_weights, weight, b)
    return out.reshape(B, S, D_OUT)


# speedup vs baseline: 1.0080x; 1.0080x over previous
import jax
import jax.numpy as jnp
from jax.experimental import pallas as pl
from jax.experimental.pallas import tpu as pltpu

E = 8
C = 512
D_IN = 2048
D_OUT = 1024
B = 1
S = 2048
T = B * S
MCH = 4
MB = T // MCH


def _fused_kernel(x_ref, cw_ref, w_ref, b_ref, out_ref, acc_s):
    i = pl.program_id(0)
    x = x_ref[0].astype(jnp.bfloat16)
    w = w_ref[0].astype(jnp.bfloat16)
    tmp = jnp.dot(x, w, preferred_element_type=jnp.float32)
    tmp = (tmp + b_ref[0]).astype(jnp.bfloat16)

    def chunk_dot(m):
        cw_m = cw_ref[m * MB:(m + 1) * MB, :].astype(jnp.bfloat16)
        return jnp.dot(cw_m, tmp, preferred_element_type=jnp.float32)

    @pl.when(i == 0)
    def _init():
        for m in range(MCH):
            acc_s[m * MB:(m + 1) * MB, :] = chunk_dot(m).astype(jnp.bfloat16)

    @pl.when(jnp.logical_and(i > 0, i < E - 1))
    def _acc():
        for m in range(MCH):
            sl = slice(m * MB, (m + 1) * MB)
            acc_s[sl, :] = (acc_s[sl, :].astype(jnp.float32)
                            + chunk_dot(m)).astype(jnp.bfloat16)

    @pl.when(i == E - 1)
    def _last():
        for m in range(MCH):
            sl = slice(m * MB, (m + 1) * MB)
            out_ref[sl, :] = acc_s[sl, :].astype(jnp.float32) + chunk_dot(m)


def kernel(inputs, combine_weights, weight, bias):
    b = bias.reshape(E, 1, D_OUT)

    out = pl.pallas_call(
        _fused_kernel,
        grid=(E,),
        in_specs=[
            pl.BlockSpec((1, C, D_IN), lambda i: (i, 0, 0)),
            pl.BlockSpec((T, C), lambda i: (0, i)),
            pl.BlockSpec((1, D_IN, D_OUT), lambda i: (i, 0, 0)),
            pl.BlockSpec((1, 1, D_OUT), lambda i: (i, 0, 0)),
        ],
        out_specs=pl.BlockSpec((T, D_OUT), lambda i: (0, 0)),
        out_shape=jax.ShapeDtypeStruct((T, D_OUT), jnp.float32),
        scratch_shapes=[pltpu.VMEM((T, D_OUT), jnp.bfloat16)],
    )(inputs, combine_weights, weight, b)
    return out.reshape(B, S, D_OUT)
